# baseline (speedup 1.0000x reference)
import functools

import jax
import jax.numpy as jnp
from jax import lax
from jax.experimental import pallas as pl
from jax.experimental.pallas import tpu as pltpu
from jax.experimental.pallas import tpu_sc as plsc

S = 13
BB = 2
C = 20
NCH = 5 + C
ROW = S * S * BB * NCH
BATCH = 256
NOBJ = 8
NC, NS, L = 2, 16, 16
NW = NC * NS
BPW = BATCH // NW
WORDS = BPW * ROW
NQ = BPW * S * S * BB
CLAMP = float(S - 1e-6)

_mesh = plsc.VectorSubcoreMesh(core_axis_name="c", subcore_axis_name="s")


@functools.partial(
    pl.kernel,
    out_type=jax.ShapeDtypeStruct((NW, L), jnp.float32),
    mesh=_mesh,
    scratch_types=[
        pltpu.VMEM((WORDS,), jnp.float32),
        pltpu.VMEM((NW * NOBJ,), jnp.float32),
        pltpu.VMEM((NW * 2,), jnp.int32),
        pltpu.VMEM((L,), jnp.float32),
        pltpu.SemaphoreType.DMA,
    ],
    compiler_params=pltpu.CompilerParams(needs_layout_passes=False),
)
def _yolo_sc(pred_hbm, box_hbm, out_hbm, pred_v, box_v, cell_v, res_v, sem):
    wid = lax.axis_index("s") * NC + lax.axis_index("c")
    pltpu.sync_copy(box_hbm.at[wid], box_v)
    pltpu.sync_copy(pred_hbm.at[wid], pred_v)

    iota = lax.iota(jnp.int32, L)
    zero = jnp.float32(0.0) * iota.astype(jnp.float32)

    geom = []
    for g in range(4):
        x1 = box_v[pl.ds(0 * 64 + g * L, L)]
        y1 = box_v[pl.ds(1 * 64 + g * L, L)]
        x2 = box_v[pl.ds(2 * 64 + g * L, L)]
        y2 = box_v[pl.ds(3 * 64 + g * L, L)]
        x = jnp.minimum(((x1 + x2) / 2.0) / 32.0, CLAMP)
        y = jnp.minimum(((y1 + y2) / 2.0) / 32.0, CLAMP)
        gxi = x.astype(jnp.int32)
        gyi = y.astype(jnp.int32)
        xoff = x - gxi.astype(jnp.float32)
        yoff = y - gyi.astype(jnp.float32)
        wn = (x2 - x1) / 416.0
        hn = (y2 - y1) / 416.0
        cellv = gyi * S + gxi
        cell_v[pl.ds(g * L, L)] = cellv
        geom.append((xoff, yoff, wn, hn, cellv))

    sparse_acc = zero
    jj = iota & 7
    for g in range(4):
        xoff, yoff, wn, hn, cellv = geom[g]
        dead = iota < 0
        for s in range(1, NOBJ):
            valid = jj < (NOBJ - s)
            idx = jnp.where(valid, g * L + iota + s, 0)
            other = plsc.load_gather(cell_v, [idx])
            dead = dead | (valid & (other == cellv))
        alive = jnp.where(dead, 0.0, 1.0)

        bl = (iota >> 3) + 2 * g
        base = bl * ROW + cellv * (2 * NCH)
        contrib = zero
        for c in range(NCH):
            pv = plsc.load_gather(pred_v, [base + c])
            if c == 0:
                d = pv - xoff
                contrib = contrib + 5.0 * (d * d)
            elif c == 1:
                d = pv - yoff
                contrib = contrib + 5.0 * (d * d)
            elif c == 2:
                d = pv - wn
                contrib = contrib + 5.0 * (d * d)
            elif c == 3:
                d = pv - hn
                contrib = contrib + 5.0 * (d * d)
            elif c == 4:
                contrib = contrib + (1.0 - 2.0 * pv)
            else:
                contrib = contrib + pv * pv
        sparse_acc = sparse_acc + alive * contrib

    def conf_body(k, acc):
        idx = (k * L + iota) * NCH + 4
        v = plsc.load_gather(pred_v, [idx])
        return acc + v * v

    conf_acc = lax.fori_loop(0, NQ // L, conf_body, zero)

    res_v[...] = conf_acc + sparse_acc
    pltpu.sync_copy(res_v, out_hbm.at[wid])


def kernel(predictions, boxes, labels):
    preds_r = predictions.reshape(NW, WORDS)
    boxes_r = (boxes.transpose(2, 0, 1)
                    .reshape(4, NW, NW * 2)
                    .transpose(1, 0, 2)
                    .reshape(NW, NW * NOBJ))
    partials = _yolo_sc(preds_r, boxes_r)
    return jnp.sum(partials) / predictions.shape[0]


# speedup vs baseline: 2.8711x; 2.8711x over previous
import functools

import jax
import jax.numpy as jnp
from jax import lax
from jax.experimental import pallas as pl
from jax.experimental.pallas import tpu as pltpu
from jax.experimental.pallas import tpu_sc as plsc

S = 13
NCELL = S * S
NCH = 25
BATCH = 256
NOBJ = 8
NC, NS, L = 2, 16, 16
NW = NC * NS
NSLOT = 6
CLAMP = float(S - 1e-6)

_mesh = plsc.VectorSubcoreMesh(core_axis_name="c", subcore_axis_name="s")


@functools.partial(
    pl.kernel,
    out_type=jax.ShapeDtypeStruct((NW, L), jnp.float32),
    mesh=_mesh,
    scratch_types=[
        pltpu.VMEM((NCH, 2, 2, 128), jnp.float32),
        pltpu.VMEM((NOBJ, 2, 4, 128), jnp.float32),
        pltpu.VMEM((NOBJ, 2, 128), jnp.int32),
        pltpu.VMEM((NOBJ, 2, 128), jnp.float32),
        pltpu.VMEM((NOBJ, 2, 128), jnp.float32),
        pltpu.VMEM((NOBJ, 2, 128), jnp.float32),
        pltpu.VMEM((NOBJ, 2, 128), jnp.float32),
        pltpu.VMEM((L,), jnp.float32),
    ],
)
def _yolo_sc(pred_hbm, box_hbm, out_hbm, cell_b, box_b, cid_s, tx_s, ty_s,
             tw_s, th_s, acc_r):
    wid = lax.axis_index("s") * NC + lax.axis_index("c")
    pltpu.sync_copy(box_hbm, box_b)

    iota = lax.iota(jnp.int32, L)
    zero = jnp.float32(0.0) * iota.astype(jnp.float32)
    acc_r[...] = zero

    def geom_body(i, _):
        j = i >> 4
        bh = (i >> 3) & 1
        bl0 = (i & 7) * L
        x1 = box_b[j, bh, 0, pl.ds(bl0, L)]
        y1 = box_b[j, bh, 1, pl.ds(bl0, L)]
        x2 = box_b[j, bh, 2, pl.ds(bl0, L)]
        y2 = box_b[j, bh, 3, pl.ds(bl0, L)]
        x = jnp.minimum(((x1 + x2) / 2.0) / 32.0, CLAMP)
        y = jnp.minimum(((y1 + y2) / 2.0) / 32.0, CLAMP)
        gxi = x.astype(jnp.int32)
        gyi = y.astype(jnp.int32)
        cid_s[j, bh, pl.ds(bl0, L)] = gyi * S + gxi
        tx_s[j, bh, pl.ds(bl0, L)] = x - gxi.astype(jnp.float32)
        ty_s[j, bh, pl.ds(bl0, L)] = y - gyi.astype(jnp.float32)
        tw_s[j, bh, pl.ds(bl0, L)] = (x2 - x1) / 416.0
        th_s[j, bh, pl.ds(bl0, L)] = (y2 - y1) / 416.0
        return 0

    lax.fori_loop(0, NOBJ * 2 * NOBJ, geom_body, 0)

    for slot in range(NSLOT):
        cid = slot * NW + wid

        @pl.when(cid < NCELL)
        def _process():
            gy = cid // S
            gx = cid % S
            pltpu.sync_copy(pred_hbm.at[gy, gx], cell_b)

            def bg_body(bg, acc):
                bh = bg >> 3
                bl0 = (bg & 7) * L
                v0 = cell_b[4, bh, 0, pl.ds(bl0, L)]
                v1 = cell_b[4, bh, 1, pl.ds(bl0, L)]
                acc = acc + v0 * v0 + v1 * v1
                mask = iota < 0
                tx = zero
                ty = zero
                tw = zero
                th = zero
                for j in range(NOBJ):
                    cj = cid_s[j, bh, pl.ds(bl0, L)]
                    m = cj == cid
                    mask = mask | m
                    tx = jnp.where(m, tx_s[j, bh, pl.ds(bl0, L)], tx)
                    ty = jnp.where(m, ty_s[j, bh, pl.ds(bl0, L)], ty)
                    tw = jnp.where(m, tw_s[j, bh, pl.ds(bl0, L)], tw)
                    th = jnp.where(m, th_s[j, bh, pl.ds(bl0, L)], th)
                coord = zero
                cls = zero
                for ch in range(NCH):
                    v = cell_b[ch, bh, 0, pl.ds(bl0, L)]
                    if ch == 0:
                        d = v - tx
                        coord = coord + d * d
                    elif ch == 1:
                        d = v - ty
                        coord = coord + d * d
                    elif ch == 2:
                        d = v - tw
                        coord = coord + d * d
                    elif ch == 3:
                        d = v - th
                        coord = coord + d * d
                    elif ch == 4:
                        conf_c = 1.0 - 2.0 * v
                    else:
                        cls = cls + v * v
                corr = 5.0 * coord + conf_c + cls
                return acc + jnp.where(mask, corr, 0.0)

            total = lax.fori_loop(0, 16, bg_body, zero)
            acc_r[...] = acc_r[...] + total

    pltpu.sync_copy(acc_r, out_hbm.at[wid])


def kernel(predictions, boxes, labels):
    pred6 = (predictions.reshape(2, 128, S, S, 2, NCH)
                        .transpose(2, 3, 5, 0, 4, 1))
    box4 = (boxes.reshape(2, 128, NOBJ, 4)
                 .transpose(2, 0, 3, 1))
    partials = _yolo_sc(pred6, box4)
    return jnp.sum(partials) / predictions.shape[0]


# speedup vs baseline: 3.2760x; 1.1410x over previous
import functools

import jax
import jax.numpy as jnp
from jax import lax
from jax.experimental import pallas as pl
from jax.experimental.pallas import tpu as pltpu
from jax.experimental.pallas import tpu_sc as plsc

S = 13
NCELL = S * S
NCH = 25
BATCH = 256
NOBJ = 8
NC, NS, L = 2, 16, 16
NW = NC * NS
NSLOT = 6
CLAMP = float(S - 1e-6)

_mesh = plsc.VectorSubcoreMesh(core_axis_name="c", subcore_axis_name="s")


@functools.partial(
    pl.kernel,
    out_type=jax.ShapeDtypeStruct((NW, L), jnp.float32),
    mesh=_mesh,
    scratch_types=[
        pltpu.VMEM((NCH, 2, 2, 128), jnp.float32),
        pltpu.VMEM((NCH, 2, 2, 128), jnp.float32),
        pltpu.VMEM((NOBJ, 2, 4, 128), jnp.float32),
        pltpu.VMEM((NOBJ, 2, 128), jnp.int32),
        pltpu.VMEM((NOBJ, 2, 128), jnp.float32),
        pltpu.VMEM((NOBJ, 2, 128), jnp.float32),
        pltpu.VMEM((NOBJ, 2, 128), jnp.float32),
        pltpu.VMEM((NOBJ, 2, 128), jnp.float32),
        pltpu.VMEM((L,), jnp.float32),
        pltpu.SemaphoreType.DMA,
        pltpu.SemaphoreType.DMA,
    ],
)
def _yolo_sc(pred_hbm, box_hbm, out_hbm, cb0, cb1, box_b, cid_s, tx_s, ty_s,
             tw_s, th_s, acc_r, sem0, sem1):
    wid = lax.axis_index("s") * NC + lax.axis_index("c")
    bufs = (cb0, cb1)
    sems = (sem0, sem1)

    def start_fetch(slot):
        cid = slot * NW + wid

        @pl.when(cid < NCELL)
        def _():
            pltpu.async_copy(pred_hbm.at[cid // S, cid % S],
                             bufs[slot % 2], sems[slot % 2])

    start_fetch(0)
    pltpu.sync_copy(box_hbm, box_b)

    iota = lax.iota(jnp.int32, L)
    zero = jnp.float32(0.0) * iota.astype(jnp.float32)
    acc_r[...] = zero

    def geom_body(i, _):
        j = i >> 4
        bh = (i >> 3) & 1
        bl0 = (i & 7) * L
        x1 = box_b[j, bh, 0, pl.ds(bl0, L)]
        y1 = box_b[j, bh, 1, pl.ds(bl0, L)]
        x2 = box_b[j, bh, 2, pl.ds(bl0, L)]
        y2 = box_b[j, bh, 3, pl.ds(bl0, L)]
        x = jnp.minimum(((x1 + x2) / 2.0) / 32.0, CLAMP)
        y = jnp.minimum(((y1 + y2) / 2.0) / 32.0, CLAMP)
        gxi = x.astype(jnp.int32)
        gyi = y.astype(jnp.int32)
        cid_s[j, bh, pl.ds(bl0, L)] = gyi * S + gxi
        tx_s[j, bh, pl.ds(bl0, L)] = x - gxi.astype(jnp.float32)
        ty_s[j, bh, pl.ds(bl0, L)] = y - gyi.astype(jnp.float32)
        tw_s[j, bh, pl.ds(bl0, L)] = (x2 - x1) / 416.0
        th_s[j, bh, pl.ds(bl0, L)] = (y2 - y1) / 416.0
        return 0

    lax.fori_loop(0, NOBJ * 2 * NOBJ, geom_body, 0)

    for slot in range(NSLOT):
        cid = slot * NW + wid
        if slot + 1 < NSLOT:
            start_fetch(slot + 1)

        @pl.when(cid < NCELL)
        def _process():
            cell_b = bufs[slot % 2]
            pltpu.make_async_copy(pred_hbm.at[cid // S, cid % S],
                                  cell_b, sems[slot % 2]).wait()

            def bg_body(bg, acc):
                bh = bg >> 3
                bl0 = (bg & 7) * L
                v0 = cell_b[4, bh, 0, pl.ds(bl0, L)]
                v1 = cell_b[4, bh, 1, pl.ds(bl0, L)]
                acc = acc + v0 * v0 + v1 * v1
                mask = iota < 0
                tx = zero
                ty = zero
                tw = zero
                th = zero
                for j in range(NOBJ):
                    cj = cid_s[j, bh, pl.ds(bl0, L)]
                    m = cj == cid
                    mask = mask | m
                    tx = jnp.where(m, tx_s[j, bh, pl.ds(bl0, L)], tx)
                    ty = jnp.where(m, ty_s[j, bh, pl.ds(bl0, L)], ty)
                    tw = jnp.where(m, tw_s[j, bh, pl.ds(bl0, L)], tw)
                    th = jnp.where(m, th_s[j, bh, pl.ds(bl0, L)], th)
                coord = zero
                cls = zero
                for ch in range(NCH):
                    v = cell_b[ch, bh, 0, pl.ds(bl0, L)]
                    if ch == 0:
                        d = v - tx
                        coord = coord + d * d
                    elif ch == 1:
                        d = v - ty
                        coord = coord + d * d
                    elif ch == 2:
                        d = v - tw
                        coord = coord + d * d
                    elif ch == 3:
                        d = v - th
                        coord = coord + d * d
                    elif ch == 4:
                        conf_c = 1.0 - 2.0 * v
                    else:
                        cls = cls + v * v
                corr = 5.0 * coord + conf_c + cls
                return acc + jnp.where(mask, corr, 0.0)

            total = lax.fori_loop(0, 16, bg_body, zero)
            acc_r[...] = acc_r[...] + total

    pltpu.sync_copy(acc_r, out_hbm.at[wid])


def kernel(predictions, boxes, labels):
    pred6 = (predictions.reshape(2, 128, S, S, 2, NCH)
                        .transpose(2, 3, 5, 0, 4, 1))
    box4 = (boxes.reshape(2, 128, NOBJ, 4)
                 .transpose(2, 0, 3, 1))
    partials = _yolo_sc(pred6, box4)
    return jnp.sum(partials) / predictions.shape[0]
